# Initial kernel scaffold; baseline (speedup 1.0000x reference)
import jax
import jax.numpy as jnp
from jax.experimental import pallas as pl


def kernel(g, x, external_input, conv2_w, conv2_b, disc_w1, disc_b1, disc_w2, disc_b2):
    raise NotImplementedError("write your pallas kernel here")


# speedup vs baseline: 21.1714x; 21.1714x over previous
import functools

import jax
import jax.numpy as jnp
from jax import lax
from jax.experimental import pallas as pl
from jax.experimental.pallas import tpu as pltpu
from jax.experimental.pallas import tpu_sc as plsc

N_NODES = 100000
N_EDGES = 3200000
HID = 16

NC = 2
NS = 16
NW = NC * NS
LANE = 128

E_ROWS = N_EDGES // LANE
ROWS_PER_W = 784
E_ROWS_PAD = NW * ROWS_PER_W
SB_DEG = 56
SB_AGG = 16

NPAD = 100352
ROWS_PER_TILE = NPAD // NS
ZCH = 392

_f32 = jnp.float32


def _sc_mesh():
  return plsc.VectorSubcoreMesh(
      core_axis_name="c", subcore_axis_name="s", num_cores=NC,
      num_subcores=NS)


def _degrees(g2p):
  @functools.partial(
      pl.kernel,
      out_type=[
          jax.ShapeDtypeStruct((NC * NPAD,), _f32),
          jax.ShapeDtypeStruct((NC * NPAD,), _f32),
      ],
      mesh=_sc_mesh(),
      compiler_params=pltpu.CompilerParams(use_tc_tiling_on_sc=False),
      scratch_types=[
          pltpu.VMEM_SHARED((NPAD,), _f32),
          pltpu.VMEM_SHARED((NPAD,), _f32),
          pltpu.VMEM((ROWS_PER_TILE,), _f32),
          pltpu.VMEM((LANE,), _f32),
          pltpu.VMEM((SB_DEG, LANE), jnp.int32),
          pltpu.VMEM((SB_DEG, LANE), jnp.int32),
      ],
  )
  def deg_kernel(g_hbm, dsrc_hbm, ddst_hbm, dsrc_s, ddst_s, zbuf, ones,
                 sidx, didx):
    c = lax.axis_index("c")
    s = lax.axis_index("s")
    w = c * NS + s

    def _fill(i, _):
      zbuf[pl.ds(i * 16, 16)] = jnp.zeros((16,), _f32)
      return ()
    lax.fori_loop(0, ROWS_PER_TILE // 16, _fill, ())
    def _fill1(i, _):
      ones[pl.ds(i * 16, 16)] = jnp.ones((16,), _f32)
      return ()
    lax.fori_loop(0, LANE // 16, _fill1, ())

    off = s * ROWS_PER_TILE
    pltpu.sync_copy(zbuf, dsrc_s.at[pl.ds(off, ROWS_PER_TILE)])
    pltpu.sync_copy(zbuf, ddst_s.at[pl.ds(off, ROWS_PER_TILE)])
    plsc.subcore_barrier()

    row0 = w * ROWS_PER_W

    def _stage(t, _):
      base = row0 + t * SB_DEG
      pltpu.sync_copy(g_hbm.at[0, pl.ds(base, SB_DEG), :], sidx)
      pltpu.sync_copy(g_hbm.at[1, pl.ds(base, SB_DEG), :], didx)

      def _chunk(j, _):
        pltpu.sync_copy(ones, dsrc_s.at[sidx.at[j]], add=True)
        pltpu.sync_copy(ones, ddst_s.at[didx.at[j]], add=True)
        return ()
      lax.fori_loop(0, SB_DEG, _chunk, ())
      return ()
    lax.fori_loop(0, ROWS_PER_W // SB_DEG, _stage, ())
    plsc.subcore_barrier()

    pltpu.sync_copy(dsrc_s.at[pl.ds(off, ROWS_PER_TILE)], zbuf)
    pltpu.sync_copy(zbuf, dsrc_hbm.at[pl.ds(c * NPAD + off, ROWS_PER_TILE)])
    pltpu.sync_copy(ddst_s.at[pl.ds(off, ROWS_PER_TILE)], zbuf)
    pltpu.sync_copy(zbuf, ddst_hbm.at[pl.ds(c * NPAD + off, ROWS_PER_TILE)])

  return deg_kernel(g2p)


def _norm_table(dsrc_p, ddst_p, xpad):
  RB = 3136
  grid = (NPAD // RB,)

  def body(ds_ref, dd_ref, x_ref, tab_ref, nd_ref):
    dsum = ds_ref[0] + ds_ref[1]
    ddum = dd_ref[0] + dd_ref[1]
    ns_ = lax.rsqrt(jnp.maximum(dsum, 1.0))
    nd_ = lax.rsqrt(jnp.maximum(ddum, 1.0))
    tab_ref[...] = jnp.maximum(x_ref[...], 0.0) * ns_
    nd_ref[...] = nd_

  return pl.pallas_call(
      body,
      grid=grid,
      in_specs=[
          pl.BlockSpec((NC, RB, 1), lambda i: (0, i, 0)),
          pl.BlockSpec((NC, RB, 1), lambda i: (0, i, 0)),
          pl.BlockSpec((RB, HID), lambda i: (i, 0)),
      ],
      out_specs=[
          pl.BlockSpec((RB, HID), lambda i: (i, 0)),
          pl.BlockSpec((RB, 1), lambda i: (i, 0)),
      ],
      out_shape=[
          jax.ShapeDtypeStruct((NPAD, HID), _f32),
          jax.ShapeDtypeStruct((NPAD, 1), _f32),
      ],
  )(dsrc_p, ddst_p, xpad)


def _aggregate(g2p, table):
  @functools.partial(
      pl.kernel,
      out_type=jax.ShapeDtypeStruct((NC, NPAD, HID), _f32),
      mesh=_sc_mesh(),
      compiler_params=pltpu.CompilerParams(use_tc_tiling_on_sc=False),
      scratch_types=[
          pltpu.VMEM_SHARED((NPAD, HID), _f32),
          pltpu.VMEM((ZCH, HID), _f32),
          pltpu.VMEM((SB_AGG, LANE), jnp.int32),
          pltpu.VMEM((SB_AGG, LANE), jnp.int32),
          pltpu.VMEM((LANE, HID), _f32),
          pltpu.VMEM((LANE, HID), _f32),
          pltpu.VMEM((LANE, HID), _f32),
          pltpu.VMEM((LANE, HID), _f32),
          pltpu.SemaphoreType.DMA,
          pltpu.SemaphoreType.DMA,
          pltpu.SemaphoreType.DMA,
          pltpu.SemaphoreType.DMA,
      ],
  )
  def agg_kernel(g_hbm, tab_hbm, out_hbm, agg_s, zbuf, sidx, didx,
                 rows0, rows1, rows2, rows3, sem0, sem1, sem2, sem3):
    c = lax.axis_index("c")
    s = lax.axis_index("s")
    w = c * NS + s

    def _fill(i, _):
      zbuf[i] = jnp.zeros((HID,), _f32)
      return ()
    lax.fori_loop(0, ZCH, _fill, ())

    off = s * ROWS_PER_TILE
    def _z(k, _):
      pltpu.sync_copy(zbuf, agg_s.at[pl.ds(off + k * ZCH, ZCH), :])
      return ()
    lax.fori_loop(0, ROWS_PER_TILE // ZCH, _z, ())
    plsc.subcore_barrier()

    row0 = w * ROWS_PER_W

    def _stage(t, _):
      base = row0 + t * SB_AGG
      pltpu.sync_copy(g_hbm.at[0, pl.ds(base, SB_AGG), :], sidx)
      pltpu.sync_copy(g_hbm.at[1, pl.ds(base, SB_AGG), :], didx)

      def _group(q, _):
        j = q * 4
        d0 = pltpu.async_copy(tab_hbm.at[sidx.at[j]], rows0, sem0)
        d1 = pltpu.async_copy(tab_hbm.at[sidx.at[j + 1]], rows1, sem1)
        d2 = pltpu.async_copy(tab_hbm.at[sidx.at[j + 2]], rows2, sem2)
        d3 = pltpu.async_copy(tab_hbm.at[sidx.at[j + 3]], rows3, sem3)
        d0.wait()
        pltpu.sync_copy(rows0, agg_s.at[didx.at[j]], add=True)
        d1.wait()
        pltpu.sync_copy(rows1, agg_s.at[didx.at[j + 1]], add=True)
        d2.wait()
        pltpu.sync_copy(rows2, agg_s.at[didx.at[j + 2]], add=True)
        d3.wait()
        pltpu.sync_copy(rows3, agg_s.at[didx.at[j + 3]], add=True)
        return ()
      lax.fori_loop(0, SB_AGG // 4, _group, ())
      return ()
    lax.fori_loop(0, ROWS_PER_W // SB_AGG, _stage, ())
    plsc.subcore_barrier()

    def _wo(k, _):
      r0 = off + k * ZCH
      pltpu.sync_copy(agg_s.at[pl.ds(r0, ZCH), :], zbuf)
      pltpu.sync_copy(zbuf, out_hbm.at[c, pl.ds(r0, ZCH), :])
      return ()
    lax.fori_loop(0, ROWS_PER_TILE // ZCH, _wo, ())

  return agg_kernel(g2p, table)


def _finish_conv(agg_p, norm_dst, conv2_w, conv2_b2d):
  RB = 3136
  grid = (NPAD // RB,)

  def body(a_ref, nd_ref, w_ref, b_ref, o_ref):
    agg = (a_ref[0] + a_ref[1]) * nd_ref[...]
    o_ref[...] = jnp.dot(agg, w_ref[...],
                         preferred_element_type=_f32) + b_ref[...]

  return pl.pallas_call(
      body,
      grid=grid,
      in_specs=[
          pl.BlockSpec((NC, RB, HID), lambda i: (0, i, 0)),
          pl.BlockSpec((RB, 1), lambda i: (i, 0)),
          pl.BlockSpec((HID, HID), lambda i: (0, 0)),
          pl.BlockSpec((1, HID), lambda i: (0, 0)),
      ],
      out_specs=pl.BlockSpec((RB, HID), lambda i: (i, 0)),
      out_shape=jax.ShapeDtypeStruct((NPAD, HID), _f32),
  )(agg_p, norm_dst, conv2_w, conv2_b2d)


def _discriminator(h3, w1, b1_2d, w2, b2_2d):
  RB = 2000
  n_rows = 2 * N_NODES
  grid = (n_rows // RB,)

  def body(x_ref, w1_ref, b1_ref, w2_ref, b2_ref, o_ref):
    h = jnp.dot(x_ref[...], w1_ref[...], preferred_element_type=_f32)
    h = jnp.maximum(h + b1_ref[...], 0.0)
    raw = jnp.dot(h, w2_ref[...], preferred_element_type=_f32) + b2_ref[...]
    m = jnp.max(raw, axis=1, keepdims=True)
    lse = m + jnp.log(jnp.sum(jnp.exp(raw - m), axis=1, keepdims=True))
    o_ref[...] = raw - lse

  return pl.pallas_call(
      body,
      grid=grid,
      in_specs=[
          pl.BlockSpec((RB, HID), lambda i: (i, 0)),
          pl.BlockSpec((HID, 8), lambda i: (0, 0)),
          pl.BlockSpec((1, 8), lambda i: (0, 0)),
          pl.BlockSpec((8, 2), lambda i: (0, 0)),
          pl.BlockSpec((1, 2), lambda i: (0, 0)),
      ],
      out_specs=pl.BlockSpec((RB, 2), lambda i: (i, 0)),
      out_shape=jax.ShapeDtypeStruct((n_rows, 2), _f32),
  )(h3, w1, b1_2d, w2, b2_2d)


def kernel(g, x, external_input, conv2_w, conv2_b, disc_w1, disc_b1,
           disc_w2, disc_b2):
  g2 = g.reshape(2, E_ROWS, LANE)
  n_pad_rows = E_ROWS_PAD - E_ROWS
  pad_idx = (N_NODES + jnp.arange(n_pad_rows * LANE, dtype=jnp.int32)
             % (NPAD - N_NODES)).reshape(1, n_pad_rows, LANE)
  pad_blk = jnp.concatenate([pad_idx, pad_idx], axis=0)
  g2p = jnp.concatenate([g2, pad_blk], axis=1)

  xpad = jnp.pad(x, ((0, NPAD - N_NODES), (0, 0)))

  dsrc_p, ddst_p = _degrees(g2p)

  table, norm_dst = _norm_table(dsrc_p.reshape(NC, NPAD, 1),
                                ddst_p.reshape(NC, NPAD, 1), xpad)

  agg_p = _aggregate(g2p, table)

  conv_out = _finish_conv(agg_p, norm_dst, conv2_w,
                          conv2_b.reshape(1, HID))

  h3 = jnp.concatenate([x, external_input], axis=0)
  logits = _discriminator(h3, disc_w1, disc_b1.reshape(1, 8),
                          disc_w2, disc_b2.reshape(1, 2))

  return (conv_out[:N_NODES], logits)
